# baseline (device time: 34075 ns/iter reference)
import jax
import jax.numpy as jnp
from jax import lax
from jax.experimental import pallas as pl
from jax.experimental.pallas import tpu as pltpu

N_DEV = 8
N_LAYERS = 3
N_STEPS = 3

STEP_MASKS = (1, 3, 4)

THIRD_SIZES = (88, 88, 80)
THIRD_OFFS = (0, 88, 176)
N_THIRDS = 3


def kernel(x, Win0, Wout0, Win1, Wout1, Win2, Wout2):
    b, d = x.shape
    n_t = N_LAYERS * N_STEPS

    def kernel_body(
        x_ref,
        win0_ref,
        wout0_ref,
        win1_ref,
        wout1_ref,
        win2_ref,
        wout2_ref,
        out_ref,
        acc_ref,
        comm_ref,
        stage_ref,
        send_sems,
        recv_sems,
    ):
        my = lax.axis_index("i")

        barrier_sem = pltpu.get_barrier_semaphore()
        for s in range(N_STEPS):
            pl.semaphore_signal(
                barrier_sem,
                inc=1,
                device_id=(my ^ STEP_MASKS[s],),
                device_id_type=pl.DeviceIdType.MESH,
            )

        acc_ref[...] = x_ref[...]
        wins = [win0_ref, win1_ref, win2_ref]
        wouts = [wout0_ref, wout1_ref, wout2_ref]

        def start_exchange(t, j):
            s = t % N_STEPS
            partner = my ^ STEP_MASKS[(s + j) % N_STEPS]
            off, sz = THIRD_OFFS[j], THIRD_SIZES[j]
            idx = t * N_THIRDS + j
            rdma = pltpu.make_async_remote_copy(
                src_ref=stage_ref.at[t, pl.ds(off, sz), :],
                dst_ref=comm_ref.at[t, pl.ds(off, sz), :],
                send_sem=send_sems.at[idx],
                recv_sem=recv_sems.at[idx],
                device_id=(partner,),
                device_id_type=pl.DeviceIdType.MESH,
            )
            rdma.start()
            return rdma

        def compute_layer_third(l, j):
            off, sz = THIRD_OFFS[j], THIRD_SIZES[j]
            h = jnp.maximum(
                jnp.dot(
                    acc_ref[pl.ds(off, sz), :],
                    wins[l][...],
                    preferred_element_type=jnp.float32,
                ),
                0.0,
            )
            acc_ref[pl.ds(off, sz), :] = jnp.dot(
                h, wouts[l][...], preferred_element_type=jnp.float32
            )

        def stage_and_start(t, j):
            off, sz = THIRD_OFFS[j], THIRD_SIZES[j]
            stage_ref[t, pl.ds(off, sz), :] = acc_ref[pl.ds(off, sz), :].astype(
                jnp.bfloat16
            )
            return start_exchange(t, j)

        inflight = {}
        for j in range(N_THIRDS):
            compute_layer_third(0, j)
        pl.semaphore_wait(barrier_sem, N_STEPS)
        for j in range(N_THIRDS):
            inflight[j] = stage_and_start(0, j)

        for l in range(N_LAYERS):
            for s in range(N_STEPS):
                t = l * N_STEPS + s
                for j in range(N_THIRDS):
                    inflight[j].wait()
                if s < N_STEPS - 1:
                    acc_ref[...] = acc_ref[...] + comm_ref[t].astype(jnp.float32)
                    stage_ref[t + 1, :, :] = acc_ref[...].astype(jnp.bfloat16)
                    for j in range(N_THIRDS):
                        inflight[j] = start_exchange(t + 1, j)
                elif l < N_LAYERS - 1:
                    acc_ref[...] = acc_ref[...] + comm_ref[t].astype(jnp.float32)
                    for j in range(N_THIRDS):
                        compute_layer_third(l + 1, j)
                        inflight[j] = stage_and_start(t + 1, j)
                else:
                    out_ref[...] = acc_ref[...] + comm_ref[t].astype(jnp.float32)

    return pl.pallas_call(
        kernel_body,
        out_shape=jax.ShapeDtypeStruct((b, d), jnp.float32),
        in_specs=[pl.BlockSpec(memory_space=pltpu.VMEM)] * 7,
        out_specs=pl.BlockSpec(memory_space=pltpu.VMEM),
        scratch_shapes=[
            pltpu.VMEM((b, d), jnp.float32),
            pltpu.VMEM((n_t, b, d), jnp.bfloat16),
            pltpu.VMEM((n_t, b, d), jnp.bfloat16),
            pltpu.SemaphoreType.DMA((n_t * N_THIRDS,)),
            pltpu.SemaphoreType.DMA((n_t * N_THIRDS,)),
        ],
        compiler_params=pltpu.CompilerParams(collective_id=0),
    )(x, Win0, Wout0, Win1, Wout1, Win2, Wout2)


# device time: 33078 ns/iter; 1.0301x vs baseline; 1.0301x over previous
import jax
import jax.numpy as jnp
from jax import lax
from jax.experimental import pallas as pl
from jax.experimental.pallas import tpu as pltpu

N_DEV = 8
N_LAYERS = 3
N_STEPS = 3

STEP_MASKS = (1, 3, 4)

THIRD_SIZES = (88, 88, 80)
THIRD_OFFS = (0, 88, 176)
N_THIRDS = 3


def kernel(x, Win0, Wout0, Win1, Wout1, Win2, Wout2):
    b, d = x.shape

    def body(
        x_ref,
        win0_ref,
        wout0_ref,
        win1_ref,
        wout1_ref,
        win2_ref,
        wout2_ref,
        out_ref,
        acc_ref,
        comm_ref,
        stage_ref,
        send_sems,
        recv_sems,
    ):
        my = lax.axis_index("i")

        barrier_sem = pltpu.get_barrier_semaphore()
        for s in range(N_STEPS):
            partner = my ^ STEP_MASKS[s]
            pl.semaphore_signal(
                barrier_sem,
                inc=1,
                device_id=(partner,),
                device_id_type=pl.DeviceIdType.MESH,
            )

        acc_ref[...] = x_ref[...]
        wins = [win0_ref, win1_ref, win2_ref]
        wouts = [wout0_ref, wout1_ref, wout2_ref]

        def start_exchange(l, s, j):
            idx = (l * N_STEPS + s) * N_THIRDS + j
            partner = my ^ STEP_MASKS[(s + j) % N_STEPS]
            off, sz = THIRD_OFFS[j], THIRD_SIZES[j]
            stage_ref[idx, pl.ds(0, sz), :] = acc_ref[
                pl.ds(off, sz), :
            ].astype(jnp.bfloat16)
            rdma = pltpu.make_async_remote_copy(
                src_ref=stage_ref.at[idx, pl.ds(0, sz), :],
                dst_ref=comm_ref.at[idx, pl.ds(0, sz), :],
                send_sem=send_sems.at[idx],
                recv_sem=recv_sems.at[idx],
                device_id=(partner,),
                device_id_type=pl.DeviceIdType.MESH,
            )
            rdma.start()
            return rdma

        def compute_layer_third(l, j):
            off, sz = THIRD_OFFS[j], THIRD_SIZES[j]
            h = jnp.maximum(
                jnp.dot(
                    acc_ref[pl.ds(off, sz), :],
                    wins[l][...],
                    preferred_element_type=jnp.float32,
                ),
                0.0,
            )
            acc_ref[pl.ds(off, sz), :] = jnp.dot(
                h, wouts[l][...], preferred_element_type=jnp.float32
            )

        inflight = {}
        for j in range(N_THIRDS):
            compute_layer_third(0, j)
        pl.semaphore_wait(barrier_sem, N_STEPS)
        for j in range(N_THIRDS):
            inflight[j] = start_exchange(0, 0, j)

        for l in range(N_LAYERS):
            for s in range(N_STEPS):
                for j in range(N_THIRDS):
                    idx = (l * N_STEPS + s) * N_THIRDS + j
                    off, sz = THIRD_OFFS[j], THIRD_SIZES[j]
                    inflight[j].wait()
                    summed = acc_ref[pl.ds(off, sz), :] + comm_ref[
                        idx, pl.ds(0, sz), :
                    ].astype(jnp.float32)
                    if s == N_STEPS - 1 and l == N_LAYERS - 1:
                        out_ref[pl.ds(off, sz), :] = summed
                    else:
                        acc_ref[pl.ds(off, sz), :] = summed
                    if s < N_STEPS - 1:
                        inflight[j] = start_exchange(l, s + 1, j)
                    elif l < N_LAYERS - 1:
                        compute_layer_third(l + 1, j)
                        inflight[j] = start_exchange(l + 1, 0, j)

    n_bufs = N_LAYERS * N_STEPS * N_THIRDS
    return pl.pallas_call(
        body,
        out_shape=jax.ShapeDtypeStruct((b, d), jnp.float32),
        in_specs=[pl.BlockSpec(memory_space=pltpu.VMEM)] * 7,
        out_specs=pl.BlockSpec(memory_space=pltpu.VMEM),
        scratch_shapes=[
            pltpu.VMEM((b, d), jnp.float32),
            pltpu.VMEM((n_bufs, max(THIRD_SIZES), d), jnp.bfloat16),
            pltpu.VMEM((n_bufs, max(THIRD_SIZES), d), jnp.bfloat16),
            pltpu.SemaphoreType.DMA((n_bufs,)),
            pltpu.SemaphoreType.DMA((n_bufs,)),
        ],
        compiler_params=pltpu.CompilerParams(collective_id=0),
    )(x, Win0, Wout0, Win1, Wout1, Win2, Wout2)
